# baseline (speedup 1.0000x reference)
import functools

import jax
import jax.numpy as jnp
from jax.experimental import pallas as pl

_BR = 512


def _mm_kernel(x_ref, w_ref, b_ref, o_ref, *, act):
    o = jnp.dot(x_ref[...], w_ref[...], preferred_element_type=jnp.float32)
    o = o + b_ref[...]
    if act == "leaky01":
        o = jnp.where(o > 0, o, 0.01 * o)
    o_ref[...] = o


def _pad_rows(a, br):
    r = a.shape[0]
    rp = -(-r // br) * br
    if rp != r:
        a = jnp.pad(a, ((0, rp - r), (0, 0)))
    return a, r


def _mm(x, w, b=None, act=None):
    R, K = x.shape
    nout = w.shape[1]
    if b is None:
        b = jnp.zeros((nout,), jnp.float32)
    xp, _ = _pad_rows(x, _BR)
    grid = xp.shape[0] // _BR
    out = pl.pallas_call(
        functools.partial(_mm_kernel, act=act),
        grid=(grid,),
        in_specs=[
            pl.BlockSpec((_BR, K), lambda i: (i, 0)),
            pl.BlockSpec((K, nout), lambda i: (0, 0)),
            pl.BlockSpec((1, nout), lambda i: (0, 0)),
        ],
        out_specs=pl.BlockSpec((_BR, nout), lambda i: (i, 0)),
        out_shape=jax.ShapeDtypeStruct((xp.shape[0], nout), jnp.float32),
    )(xp, w, b.reshape(1, nout))
    return out[:R]


def _gru_kernel(x_ref, h_ref, wih_ref, whh_ref, bih_ref, bhh_ref, o_ref, *, relu, d):
    x = x_ref[...]
    h = h_ref[...]
    gi = jnp.dot(x, wih_ref[...], preferred_element_type=jnp.float32) + bih_ref[...]
    gh = jnp.dot(h, whh_ref[...], preferred_element_type=jnp.float32) + bhh_ref[...]
    ir, iz, inn = gi[:, :d], gi[:, d:2 * d], gi[:, 2 * d:]
    hr, hz, hn = gh[:, :d], gh[:, d:2 * d], gh[:, 2 * d:]
    r = jax.nn.sigmoid(ir + hr)
    z = jax.nn.sigmoid(iz + hz)
    c = jnp.tanh(inn + r * hn)
    o = (1.0 - z) * c + z * h
    if relu:
        o = jnp.maximum(o, 0.0)
    o_ref[...] = o


def _gru(x, h, wih, whh, bih, bhh, relu):
    R, d = x.shape
    xp, _ = _pad_rows(x, _BR)
    hp, _ = _pad_rows(h, _BR)
    grid = xp.shape[0] // _BR
    out = pl.pallas_call(
        functools.partial(_gru_kernel, relu=relu, d=d),
        grid=(grid,),
        in_specs=[
            pl.BlockSpec((_BR, d), lambda i: (i, 0)),
            pl.BlockSpec((_BR, d), lambda i: (i, 0)),
            pl.BlockSpec((d, 3 * d), lambda i: (0, 0)),
            pl.BlockSpec((d, 3 * d), lambda i: (0, 0)),
            pl.BlockSpec((1, 3 * d), lambda i: (0, 0)),
            pl.BlockSpec((1, 3 * d), lambda i: (0, 0)),
        ],
        out_specs=pl.BlockSpec((_BR, d), lambda i: (i, 0)),
        out_shape=jax.ShapeDtypeStruct((xp.shape[0], d), jnp.float32),
    )(xp, hp, wih, whh, bih.reshape(1, 3 * d), bhh.reshape(1, 3 * d))
    return out[:R]


def _seg_softmax(a, idx, num):
    m = jax.ops.segment_max(a, idx, num_segments=num)
    m = jnp.where(jnp.isfinite(m), m, 0.0)
    e = jnp.exp(a - m[idx])
    s = jax.ops.segment_sum(e, idx, num_segments=num)
    return e / (s[idx] + 1e-16)


def _elu(v):
    return jnp.where(v > 0, v, jnp.expm1(v))


def kernel(x, edge_index, edge_attr, batch, params):
    p = params
    n = x.shape[0]
    d = p["lin1_W"].shape[0]
    g = p["molgru_Wih"].shape[0]
    G = 512

    xf = p["x_emb1"][x[:, 0]] + p["x_emb2"][x[:, 1]]
    xf = _mm(xf, p["lin1_W"], p["lin1_b"], act="leaky01")

    loops = jnp.arange(n, dtype=edge_index.dtype)
    src = jnp.concatenate([edge_index[0], loops])
    dst = jnp.concatenate([edge_index[1], loops])
    sl0 = jnp.full((n,), 4, edge_attr.dtype)
    sl1 = jnp.zeros((n,), edge_attr.dtype)
    ea0 = jnp.concatenate([edge_attr[:, 0], sl0])
    ea1 = jnp.concatenate([edge_attr[:, 1], sl1])
    eemb = p["g_edge_emb1"][ea0] + p["g_edge_emb2"][ea1]

    cat = jnp.concatenate([xf[src], eemb], axis=-1)
    m = _mm(cat, p["g_lin1_W"], act="leaky01")
    al = (m * p["g_att_l"]).sum(-1) + (xf * p["g_att_r"]).sum(-1)[dst]
    al = jnp.where(al > 0, al, 0.01 * al)
    al = _seg_softmax(al, dst, n)
    m2 = _mm(m, p["g_lin2_W"])
    h = jax.ops.segment_sum(m2 * al[:, None], dst, num_segments=n) + p["g_bias"]
    h = _elu(h)
    xf = _gru(h, xf, p["gru0_Wih"], p["gru0_Whh"], p["gru0_bih"], p["gru0_bhh"], relu=True)

    s2, d2 = edge_index[0], edge_index[1]
    for l in range(1, 5):
        xl = _mm(xf, p["gat%d_W" % l])
        asr = (xl * p["gat%d_asrc" % l]).sum(-1)
        ads = (xl * p["gat%d_adst" % l]).sum(-1)
        a = asr[s2] + ads[d2]
        a = jnp.where(a > 0, a, 0.2 * a)
        a = _seg_softmax(a, d2, n)
        h = jax.ops.segment_sum(xl[s2] * a[:, None], d2, num_segments=n)
        h = _elu(h + p["gat%d_b" % l])
        xf = _gru(h, xf, p["gru%d_Wih" % l], p["gru%d_Whh" % l],
                  p["gru%d_bih" % l], p["gru%d_bhh" % l], relu=True)

    out = jnp.maximum(jax.ops.segment_sum(xf, batch, num_segments=G), 0.0)

    xs = _mm(xf, p["mol_W"])
    xs_dot = (xs * p["mol_asrc"]).sum(-1)
    for _ in range(3):
        xd = _mm(out, p["mol_W"])
        a = xs_dot + (xd * p["mol_adst"]).sum(-1)[batch]
        a = jnp.where(a > 0, a, 0.2 * a)
        a = _seg_softmax(a, batch, G)
        h = jax.ops.segment_sum(xs * a[:, None], batch, num_segments=G) + p["mol_b"]
        h = _elu(h)
        out = _gru(h, out, p["molgru_Wih"], p["molgru_Whh"],
                   p["molgru_bih"], p["molgru_bhh"], relu=True)

    w2 = jnp.pad(p["lin2_W"], ((0, 0), (0, d - 1)))
    b2 = jnp.pad(p["lin2_b"], (0, d - 1))
    pred = _mm(out, w2, b2)[:, :1]
    return (out, pred)


# speedup vs baseline: 1.0493x; 1.0493x over previous
import functools

import jax
import jax.numpy as jnp
from jax.experimental import pallas as pl

_BR = 512


def _mm_kernel(x_ref, w_ref, b_ref, o_ref, *, act):
    o = jnp.dot(x_ref[...], w_ref[...], preferred_element_type=jnp.float32)
    o = o + b_ref[...]
    if act == "leaky01":
        o = jnp.where(o > 0, o, 0.01 * o)
    o_ref[...] = o


def _pad_rows(a, br):
    r = a.shape[0]
    rp = -(-r // br) * br
    if rp != r:
        a = jnp.pad(a, ((0, rp - r), (0, 0)))
    return a, r


def _mm(x, w, b=None, act=None):
    R, K = x.shape
    nout = w.shape[1]
    if b is None:
        b = jnp.zeros((nout,), jnp.float32)
    xp, _ = _pad_rows(x, _BR)
    grid = xp.shape[0] // _BR
    out = pl.pallas_call(
        functools.partial(_mm_kernel, act=act),
        grid=(grid,),
        in_specs=[
            pl.BlockSpec((_BR, K), lambda i: (i, 0)),
            pl.BlockSpec((K, nout), lambda i: (0, 0)),
            pl.BlockSpec((1, nout), lambda i: (0, 0)),
        ],
        out_specs=pl.BlockSpec((_BR, nout), lambda i: (i, 0)),
        out_shape=jax.ShapeDtypeStruct((xp.shape[0], nout), jnp.float32),
    )(xp, w, b.reshape(1, nout))
    return out[:R]


def _gru_kernel(x_ref, h_ref, wih_ref, whh_ref, bih_ref, bhh_ref, o_ref, *, relu, d):
    x = x_ref[...]
    h = h_ref[...]
    gi = jnp.dot(x, wih_ref[...], preferred_element_type=jnp.float32) + bih_ref[...]
    gh = jnp.dot(h, whh_ref[...], preferred_element_type=jnp.float32) + bhh_ref[...]
    ir, iz, inn = gi[:, :d], gi[:, d:2 * d], gi[:, 2 * d:]
    hr, hz, hn = gh[:, :d], gh[:, d:2 * d], gh[:, 2 * d:]
    r = jax.nn.sigmoid(ir + hr)
    z = jax.nn.sigmoid(iz + hz)
    c = jnp.tanh(inn + r * hn)
    o = (1.0 - z) * c + z * h
    if relu:
        o = jnp.maximum(o, 0.0)
    o_ref[...] = o


def _gru(x, h, wih, whh, bih, bhh, relu):
    R, d = x.shape
    xp, _ = _pad_rows(x, _BR)
    hp, _ = _pad_rows(h, _BR)
    grid = xp.shape[0] // _BR
    out = pl.pallas_call(
        functools.partial(_gru_kernel, relu=relu, d=d),
        grid=(grid,),
        in_specs=[
            pl.BlockSpec((_BR, d), lambda i: (i, 0)),
            pl.BlockSpec((_BR, d), lambda i: (i, 0)),
            pl.BlockSpec((d, 3 * d), lambda i: (0, 0)),
            pl.BlockSpec((d, 3 * d), lambda i: (0, 0)),
            pl.BlockSpec((1, 3 * d), lambda i: (0, 0)),
            pl.BlockSpec((1, 3 * d), lambda i: (0, 0)),
        ],
        out_specs=pl.BlockSpec((_BR, d), lambda i: (i, 0)),
        out_shape=jax.ShapeDtypeStruct((xp.shape[0], d), jnp.float32),
    )(xp, hp, wih, whh, bih.reshape(1, 3 * d), bhh.reshape(1, 3 * d))
    return out[:R]


def _seg_softmax(a, idx, num):
    m = jax.ops.segment_max(a, idx, num_segments=num)
    m = jnp.where(jnp.isfinite(m), m, 0.0)
    e = jnp.exp(a - m[idx])
    s = jax.ops.segment_sum(e, idx, num_segments=num)
    return e / (s[idx] + 1e-16)


def _elu(v):
    return jnp.where(v > 0, v, jnp.expm1(v))


def kernel(x, edge_index, edge_attr, batch, params):
    p = params
    n = x.shape[0]
    d = p["lin1_W"].shape[0]
    G = 512

    xf = p["x_emb1"][x[:, 0]] + p["x_emb2"][x[:, 1]]
    xf = _mm(xf, p["lin1_W"], p["lin1_b"], act="leaky01")

    loops = jnp.arange(n, dtype=edge_index.dtype)
    src = jnp.concatenate([edge_index[0], loops])
    dst = jnp.concatenate([edge_index[1], loops])
    sl0 = jnp.full((n,), 4, edge_attr.dtype)
    sl1 = jnp.zeros((n,), edge_attr.dtype)
    ea0 = jnp.concatenate([edge_attr[:, 0], sl0])
    ea1 = jnp.concatenate([edge_attr[:, 1], sl1])

    w_top, w_bot = p["g_lin1_W"][:d], p["g_lin1_W"][d:]
    xfw = _mm(xf, w_top)
    tbl = (p["g_edge_emb1"][:, None, :] + p["g_edge_emb2"][None, :, :]).reshape(18, d)
    tblw = _mm(tbl, w_bot)
    combo = ea0 * 3 + ea1
    m = xfw[src] + tblw[combo]
    m = jnp.where(m > 0, m, 0.01 * m)
    al = (m * p["g_att_l"]).sum(-1) + (xf * p["g_att_r"]).sum(-1)[dst]
    al = jnp.where(al > 0, al, 0.01 * al)
    al = _seg_softmax(al, dst, n)
    msum = jax.ops.segment_sum(m * al[:, None], dst, num_segments=n)
    h = _mm(msum, p["g_lin2_W"]) + p["g_bias"]
    h = _elu(h)
    xf = _gru(h, xf, p["gru0_Wih"], p["gru0_Whh"], p["gru0_bih"], p["gru0_bhh"], relu=True)

    s2, d2 = edge_index[0], edge_index[1]
    for l in range(1, 5):
        xl = _mm(xf, p["gat%d_W" % l])
        asr = (xl * p["gat%d_asrc" % l]).sum(-1)
        ads = (xl * p["gat%d_adst" % l]).sum(-1)
        a = asr[s2] + ads[d2]
        a = jnp.where(a > 0, a, 0.2 * a)
        a = _seg_softmax(a, d2, n)
        h = jax.ops.segment_sum(xl[s2] * a[:, None], d2, num_segments=n)
        h = _elu(h + p["gat%d_b" % l])
        xf = _gru(h, xf, p["gru%d_Wih" % l], p["gru%d_Whh" % l],
                  p["gru%d_bih" % l], p["gru%d_bhh" % l], relu=True)

    out = jnp.maximum(jax.ops.segment_sum(xf, batch, num_segments=G), 0.0)

    xs = _mm(xf, p["mol_W"])
    xs_dot = (xs * p["mol_asrc"]).sum(-1)
    for _ in range(3):
        xd = _mm(out, p["mol_W"])
        a = xs_dot + (xd * p["mol_adst"]).sum(-1)[batch]
        a = jnp.where(a > 0, a, 0.2 * a)
        a = _seg_softmax(a, batch, G)
        h = jax.ops.segment_sum(xs * a[:, None], batch, num_segments=G) + p["mol_b"]
        h = _elu(h)
        out = _gru(h, out, p["molgru_Wih"], p["molgru_Whh"],
                   p["molgru_bih"], p["molgru_bhh"], relu=True)

    w2 = jnp.pad(p["lin2_W"], ((0, 0), (0, d - 1)))
    b2 = jnp.pad(p["lin2_b"], (0, d - 1))
    pred = _mm(out, w2, b2)[:, :1]
    return (out, pred)
